# Initial kernel scaffold; baseline (speedup 1.0000x reference)
import jax
import jax.numpy as jnp
from jax.experimental import pallas as pl


def kernel(x, edge_index, W, a):
    raise NotImplementedError("write your pallas kernel here")


# speedup vs baseline: 1.9395x; 1.9395x over previous
import functools

import jax
import jax.numpy as jnp
from jax import lax
from jax.experimental import pallas as pl
from jax.experimental.pallas import tpu as pltpu
from jax.experimental.pallas import tpu_sc as plsc

N = 10000
E = 160000
DF = 128
RPAD = 10112
NPAD = 10016
SIZE = RPAD * (N + 1)

NW = 32
CHUNK = 128
E_PAD = 163840
NCHUNK = E_PAD // (NW * CHUNK)

BLK = 80
NBLK = N // BLK


def _st_body(x_ref, w_ref, at_ref, s_ref, t_ref):
    wx = lax.dot_general(x_ref[...], w_ref[...], (((1,), (1,)), ((), ())),
                         preferred_element_type=jnp.float32)
    a1 = at_ref[0, pl.ds(0, DF)]
    a2 = at_ref[0, pl.ds(DF, DF)]
    s = jnp.sum(wx * a1[None, :], axis=1)
    t = jnp.sum(wx * a2[None, :], axis=1)
    s_ref[0, pl.ds(0, N)] = s
    t_ref[0, pl.ds(0, N)] = t
    s_ref[0, pl.ds(N, NPAD - N)] = jnp.zeros((NPAD - N,), jnp.float32)
    t_ref[0, pl.ds(N, NPAD - N)] = jnp.zeros((NPAD - N,), jnp.float32)


_sc_mesh = plsc.VectorSubcoreMesh(core_axis_name="c", subcore_axis_name="s")


@functools.partial(
    pl.kernel,
    out_type=(),
    mesh=_sc_mesh,
    scratch_types=[
        pltpu.VMEM((NCHUNK, CHUNK), jnp.int32),
        pltpu.VMEM((NCHUNK, CHUNK), jnp.int32),
        pltpu.VMEM((CHUNK,), jnp.float32),
        pltpu.VMEM((CHUNK,), jnp.float32),
        pltpu.VMEM((CHUNK,), jnp.float32),
        pltpu.VMEM((CHUNK,), jnp.int32),
        pltpu.SemaphoreType.DMA,
        pltpu.SemaphoreType.DMA,
    ],
)
def _sc_scatter(src_hbm, dst_hbm, s_hbm, t_hbm, buf_ref,
                srcv, dstv, sv, tv, cv, fv, sem1, sem2):
    cid = lax.axis_index("c")
    sid = lax.axis_index("s")
    wid = sid * 2 + cid
    base_row = wid * NCHUNK

    pltpu.sync_copy(src_hbm.at[pl.ds(base_row, NCHUNK)], srcv)
    pltpu.sync_copy(dst_hbm.at[pl.ds(base_row, NCHUNK)], dstv)

    def chunk_body(c, carry):
        cp1 = pltpu.async_copy(s_hbm.at[srcv.at[c]], sv, sem1)
        cp2 = pltpu.async_copy(t_hbm.at[dstv.at[c]], tv, sem2)
        cp1.wait()
        cp2.wait()
        for i in range(CHUNK // 16):
            sl = pl.ds(i * 16, 16)
            z = sv[sl] + tv[sl]
            zlr = jnp.where(z >= 0.0, z, 0.1 * z)
            cv[sl] = jnp.exp(zlr)
            fv[sl] = srcv[c, sl] * RPAD + dstv[c, sl]
        pltpu.sync_copy(cv, buf_ref.at[fv])
        return carry

    lax.fori_loop(0, NCHUNK, chunk_body, 0)


def _norm_body(buf_ref, out_ref):
    g = pl.program_id(0)
    blk = buf_ref[...]
    rs = jnp.sum(blk, axis=1)
    fix = (rs == 0.0).astype(jnp.float32)
    inv = 1.0 / (rs + fix)
    row_ids = lax.broadcasted_iota(jnp.int32, (BLK, N), 0) + g * BLK
    col_ids = lax.broadcasted_iota(jnp.int32, (BLK, N), 1)
    dmask = (col_ids == row_ids).astype(jnp.float32)
    core = lax.slice(blk, (0, 0), (BLK, N))
    out_ref[...] = (core + dmask * fix[:, None]) * inv[:, None]


def kernel(x, edge_index, W, a):
    at2d = a.reshape(1, 2 * DF)
    s2d, t2d = pl.pallas_call(
        _st_body,
        out_shape=[jax.ShapeDtypeStruct((1, NPAD), jnp.float32),
                   jax.ShapeDtypeStruct((1, NPAD), jnp.float32)],
    )(x, W, at2d)
    s1d = s2d.reshape(NPAD)
    t1d = t2d.reshape(NPAD)

    src = edge_index[0].astype(jnp.int32)
    dst = edge_index[1].astype(jnp.int32)
    npad = E_PAD - E
    src_p = jnp.concatenate([src, jnp.full((npad,), N, jnp.int32)])
    dst_p = jnp.concatenate([dst, jnp.zeros((npad,), jnp.int32)])
    src_p = src_p.reshape(E_PAD // CHUNK, CHUNK)
    dst_p = dst_p.reshape(E_PAD // CHUNK, CHUNK)

    buf_ref = jax.new_ref(jnp.zeros((SIZE,), jnp.float32))
    _sc_scatter(src_p, dst_p, s1d, t1d, buf_ref)
    dense = buf_ref[...].reshape(N + 1, RPAD)

    out = pl.pallas_call(
        _norm_body,
        grid=(NBLK,),
        in_specs=[pl.BlockSpec((BLK, RPAD), lambda g: (g, 0))],
        out_specs=pl.BlockSpec((BLK, N), lambda g: (g, 0)),
        out_shape=jax.ShapeDtypeStruct((N, N), jnp.float32),
    )(dense)
    return out
